# baseline (speedup 1.0000x reference)
import functools

import jax
import jax.numpy as jnp
from jax.experimental import pallas as pl

N_TOK = 16384
EMBED = 64
N_EXP = 16
LABEL = 64
HIDDEN = EMBED // 2

BLOCK = 1024


def _moe_block(x_ref, noise_ref, w_gate_ref, w_noise_ref, w1_ref, b1_ref,
               fc2_w_ref, fc2_b_ref, out_ref):
    x = x_ref[...]
    f32 = jnp.float32

    clean = jnp.dot(x, w_gate_ref[...], preferred_element_type=f32)
    raw = jnp.dot(x, w_noise_ref[...], preferred_element_type=f32)
    noise_std = jax.nn.softplus(raw) + 1e-2
    logits = clean + noise_ref[...] * noise_std

    col = jax.lax.broadcasted_iota(jnp.int32, logits.shape, 1)
    big = jnp.int32(N_EXP)
    v1 = jnp.max(logits, axis=1, keepdims=True)
    idx1 = jnp.min(jnp.where(logits == v1, col, big), axis=1, keepdims=True)
    masked = jnp.where(col == idx1, -jnp.inf, logits)
    v2 = jnp.max(masked, axis=1, keepdims=True)
    idx2 = jnp.min(jnp.where(masked == v2, col, big), axis=1, keepdims=True)
    e2 = jnp.exp(v2 - v1)
    g1 = 1.0 / (1.0 + e2)
    g2 = e2 / (1.0 + e2)
    gates = jnp.where(col == idx1, g1, jnp.where(col == idx2, g2, 0.0))

    h = jnp.tanh(jnp.dot(x, w1_ref[...], preferred_element_type=f32)
                 + b1_ref[...])

    dn = (((1,), (1,)), ((), ()))
    for e in range(N_EXP):
        ge = gates[:, e:e + 1]
        he = h[:, e * HIDDEN:(e + 1) * HIDDEN] * ge
        oe = jax.lax.dot_general(he, fc2_w_ref[e], dn,
                                 preferred_element_type=f32)
        out_ref[:, e, :] = oe + ge * fc2_b_ref[e][None, :]


@jax.jit
def kernel(x, noise, w_gate, w_noise, fc1_w, fc1_b, fc2_w, fc2_b):
    w1 = fc1_w.reshape(N_EXP * HIDDEN, EMBED).T
    b1 = fc1_b.reshape(1, N_EXP * HIDDEN)

    grid = (N_TOK // BLOCK,)
    out = pl.pallas_call(
        _moe_block,
        grid=grid,
        in_specs=[
            pl.BlockSpec((BLOCK, EMBED), lambda i: (i, 0)),
            pl.BlockSpec((BLOCK, N_EXP), lambda i: (i, 0)),
            pl.BlockSpec((EMBED, N_EXP), lambda i: (0, 0)),
            pl.BlockSpec((EMBED, N_EXP), lambda i: (0, 0)),
            pl.BlockSpec((EMBED, N_EXP * HIDDEN), lambda i: (0, 0)),
            pl.BlockSpec((1, N_EXP * HIDDEN), lambda i: (0, 0)),
            pl.BlockSpec((N_EXP, LABEL, HIDDEN), lambda i: (0, 0, 0)),
            pl.BlockSpec((N_EXP, LABEL), lambda i: (0, 0)),
        ],
        out_specs=pl.BlockSpec((BLOCK, N_EXP, LABEL), lambda i: (i, 0, 0)),
        out_shape=jax.ShapeDtypeStruct((N_TOK, N_EXP, LABEL), jnp.float32),
    )(x, noise, w_gate, w_noise, w1, b1, fc2_w, fc2_b)
    return out


# speedup vs baseline: 2.1043x; 2.1043x over previous
import jax
import jax.numpy as jnp
from jax.experimental import pallas as pl

N_TOK = 16384
EMBED = 64
N_EXP = 16
LABEL = 64
HIDDEN = EMBED // 2

BLOCK = 1024
GROUPS = 4
EPG = N_EXP // GROUPS
GK = EPG * HIDDEN
GN = EPG * LABEL


def _moe_block(x_ref, noise_ref, w_gate_ref, w_noise_ref, w1_ref, b1_ref,
               w2_ref, b2_ref, out_ref):
    x = x_ref[...]
    f32 = jnp.float32

    clean = jnp.dot(x, w_gate_ref[...], preferred_element_type=f32)
    raw = jnp.dot(x, w_noise_ref[...], preferred_element_type=f32)
    noise_std = jax.nn.softplus(raw) + 1e-2
    logits = clean + noise_ref[...] * noise_std

    col = jax.lax.broadcasted_iota(jnp.int32, logits.shape, 1)
    big = jnp.int32(N_EXP)
    v1 = jnp.max(logits, axis=1, keepdims=True)
    idx1 = jnp.min(jnp.where(logits == v1, col, big), axis=1, keepdims=True)
    masked = jnp.where(col == idx1, -jnp.inf, logits)
    v2 = jnp.max(masked, axis=1, keepdims=True)
    idx2 = jnp.min(jnp.where(masked == v2, col, big), axis=1, keepdims=True)
    e2 = jnp.exp(v2 - v1)
    g1 = 1.0 / (1.0 + e2)
    g2 = e2 / (1.0 + e2)

    xb = x.astype(jnp.bfloat16)
    h = jnp.tanh(jnp.dot(xb, w1_ref[...], preferred_element_type=f32)
                 + b1_ref[...])
    hb = h.astype(jnp.bfloat16)

    for j in range(GROUPS):
        oj = jnp.dot(hb[:, j * GK:(j + 1) * GK], w2_ref[j],
                     preferred_element_type=f32)
        oj = oj + b2_ref[:, j * GN:(j + 1) * GN]
        ecol = jax.lax.broadcasted_iota(jnp.int32, oj.shape, 1) // LABEL \
            + j * EPG
        gcol = jnp.where(ecol == idx1, g1,
                         jnp.where(ecol == idx2, g2, 0.0))
        out_ref[:, j * GN:(j + 1) * GN] = oj * gcol


@jax.jit
def kernel(x, noise, w_gate, w_noise, fc1_w, fc1_b, fc2_w, fc2_b):
    w1 = fc1_w.reshape(N_EXP * HIDDEN, EMBED).T.astype(jnp.bfloat16)
    b1 = fc1_b.reshape(1, N_EXP * HIDDEN)

    w2t = jnp.transpose(fc2_w, (0, 2, 1))
    eye = jnp.eye(EPG, dtype=fc2_w.dtype)
    w2g = jnp.einsum('ab,gahl->gahbl', eye,
                     w2t.reshape(GROUPS, EPG, HIDDEN, LABEL))
    w2 = w2g.reshape(GROUPS, GK, GN).astype(jnp.bfloat16)
    b2 = fc2_b.reshape(1, N_EXP * LABEL)

    grid = (N_TOK // BLOCK,)
    out = pl.pallas_call(
        _moe_block,
        grid=grid,
        in_specs=[
            pl.BlockSpec((BLOCK, EMBED), lambda i: (i, 0)),
            pl.BlockSpec((BLOCK, N_EXP), lambda i: (i, 0)),
            pl.BlockSpec((EMBED, N_EXP), lambda i: (0, 0)),
            pl.BlockSpec((EMBED, N_EXP), lambda i: (0, 0)),
            pl.BlockSpec((EMBED, N_EXP * HIDDEN), lambda i: (0, 0)),
            pl.BlockSpec((1, N_EXP * HIDDEN), lambda i: (0, 0)),
            pl.BlockSpec((GROUPS, GK, GN), lambda i: (0, 0, 0)),
            pl.BlockSpec((1, N_EXP * LABEL), lambda i: (0, 0)),
        ],
        out_specs=pl.BlockSpec((BLOCK, N_EXP * LABEL), lambda i: (i, 0)),
        out_shape=jax.ShapeDtypeStruct((N_TOK, N_EXP * LABEL), jnp.float32),
    )(x, noise, w_gate, w_noise, w1, b1, w2, b2)
    return out.reshape(N_TOK, N_EXP, LABEL)


# speedup vs baseline: 2.1347x; 1.0144x over previous
import jax
import jax.numpy as jnp
from jax.experimental import pallas as pl

N_TOK = 16384
EMBED = 64
N_EXP = 16
LABEL = 64
HIDDEN = EMBED // 2

BLOCK = 2048
GROUPS = 4
EPG = N_EXP // GROUPS
GK = EPG * HIDDEN
GN = EPG * LABEL


def _moe_block(x_ref, noise_ref, w_gate_ref, w_noise_ref, w1_ref, b1_ref,
               w2_ref, b2_ref, out_ref):
    x = x_ref[...]
    f32 = jnp.float32

    clean = jnp.dot(x, w_gate_ref[...], preferred_element_type=f32)
    raw = jnp.dot(x, w_noise_ref[...], preferred_element_type=f32)
    noise_std = jax.nn.softplus(raw) + 1e-2
    logits = clean + noise_ref[...] * noise_std

    col = jax.lax.broadcasted_iota(jnp.int32, logits.shape, 1)
    big = jnp.int32(N_EXP)
    v1 = jnp.max(logits, axis=1, keepdims=True)
    idx1 = jnp.min(jnp.where(logits == v1, col, big), axis=1, keepdims=True)
    masked = jnp.where(col == idx1, -jnp.inf, logits)
    v2 = jnp.max(masked, axis=1, keepdims=True)
    idx2 = jnp.min(jnp.where(masked == v2, col, big), axis=1, keepdims=True)
    e2 = jnp.exp(v2 - v1)
    g1 = 1.0 / (1.0 + e2)
    g2 = e2 / (1.0 + e2)

    xb = x.astype(jnp.bfloat16)
    h = jnp.tanh(jnp.dot(xb, w1_ref[...], preferred_element_type=f32)
                 + b1_ref[...])
    hb = h.astype(jnp.bfloat16)

    for j in range(GROUPS):
        oj = jnp.dot(hb[:, j * GK:(j + 1) * GK], w2_ref[j],
                     preferred_element_type=f32)
        oj = oj + b2_ref[:, j * GN:(j + 1) * GN]
        ecol = jax.lax.broadcasted_iota(jnp.int32, oj.shape, 1) // LABEL \
            + j * EPG
        gcol = jnp.where(ecol == idx1, g1,
                         jnp.where(ecol == idx2, g2, 0.0))
        out_ref[:, j * GN:(j + 1) * GN] = oj * gcol


@jax.jit
def kernel(x, noise, w_gate, w_noise, fc1_w, fc1_b, fc2_w, fc2_b):
    w1 = fc1_w.reshape(N_EXP * HIDDEN, EMBED).T.astype(jnp.bfloat16)
    b1 = fc1_b.reshape(1, N_EXP * HIDDEN)

    w2t = jnp.transpose(fc2_w, (0, 2, 1))
    eye = jnp.eye(EPG, dtype=fc2_w.dtype)
    w2g = jnp.einsum('ab,gahl->gahbl', eye,
                     w2t.reshape(GROUPS, EPG, HIDDEN, LABEL))
    w2 = w2g.reshape(GROUPS, GK, GN).astype(jnp.bfloat16)
    b2 = fc2_b.reshape(1, N_EXP * LABEL)

    grid = (N_TOK // BLOCK,)
    out = pl.pallas_call(
        _moe_block,
        grid=grid,
        in_specs=[
            pl.BlockSpec((BLOCK, EMBED), lambda i: (i, 0)),
            pl.BlockSpec((BLOCK, N_EXP), lambda i: (i, 0)),
            pl.BlockSpec((EMBED, N_EXP), lambda i: (0, 0)),
            pl.BlockSpec((EMBED, N_EXP), lambda i: (0, 0)),
            pl.BlockSpec((EMBED, N_EXP * HIDDEN), lambda i: (0, 0)),
            pl.BlockSpec((1, N_EXP * HIDDEN), lambda i: (0, 0)),
            pl.BlockSpec((GROUPS, GK, GN), lambda i: (0, 0, 0)),
            pl.BlockSpec((1, N_EXP * LABEL), lambda i: (0, 0)),
        ],
        out_specs=pl.BlockSpec((BLOCK, N_EXP * LABEL), lambda i: (i, 0)),
        out_shape=jax.ShapeDtypeStruct((N_TOK, N_EXP * LABEL), jnp.float32),
    )(x, noise, w_gate, w_noise, w1, b1, w2, b2)
    return out.reshape(N_TOK, N_EXP, LABEL)
